# Initial kernel scaffold; baseline (speedup 1.0000x reference)
import jax
import jax.numpy as jnp
from jax.experimental import pallas as pl


def kernel(x, edge_index, W):
    raise NotImplementedError("write your pallas kernel here")


# speedup vs baseline: 14.9956x; 14.9956x over previous
import functools

import jax
import jax.numpy as jnp
from jax import lax
from jax.experimental import pallas as pl
from jax.experimental.pallas import tpu as pltpu
from jax.experimental.pallas import tpu_sc as plsc

CH = 64
KB = 16
NS = 16
NC = 2
LANES = 16


def _gcn(x, edge_index, W):
  N, D_in = x.shape
  D_out = W.shape[1]
  H = D_out // 2
  E = edge_index.shape[1]

  unit = NC * NS * 8 * CH
  EP = ((E + unit - 1) // unit) * unit
  CROWS = EP // CH
  A_CR = CROWS // (NC * NS)
  C_CR = CROWS // NS
  NPAD = ((N + 64 + 127) // 128) * 128
  RPT = NPAD // NS

  row = edge_index[0].astype(jnp.int32)
  col = edge_index[1].astype(jnp.int32)
  pad_i = jnp.arange(EP - E, dtype=jnp.int32)
  row2d = jnp.concatenate([row, N + (pad_i % 64)]).reshape(CROWS, CH)
  col2d = jnp.concatenate([col, pad_i % N]).reshape(CROWS, CH)

  mesh = plsc.VectorSubcoreMesh(core_axis_name="c", subcore_axis_name="s",
                                num_cores=NC, num_subcores=NS)

  @functools.partial(
      pl.kernel,
      out_type=[jax.ShapeDtypeStruct((NPAD, LANES), jnp.float32)] * 2,
      mesh=mesh,
      scratch_types=[
          pltpu.VMEM((A_CR, CH), jnp.int32),
          pltpu.VMEM((CH, LANES), jnp.float32),
          pltpu.VMEM((CH, LANES), jnp.float32),
          pltpu.VMEM_SHARED((NPAD, LANES), jnp.float32),
      ],
  )
  def deg_kernel(row_hbm, deg0_hbm, deg1_hbm, idx_v, ones_v, zb_v, acc_sh):
    c = lax.axis_index("c")
    s = lax.axis_index("s")
    r0 = s * RPT

    @pl.loop(0, CH)
    def _(i):
      ones_v[i, :] = jnp.ones((LANES,), jnp.float32)
      zb_v[i, :] = jnp.zeros((LANES,), jnp.float32)

    @pl.loop(0, RPT // CH)
    def _(k):
      pltpu.sync_copy(zb_v, acc_sh.at[pl.ds(r0 + k * CH, CH)])
    if RPT % CH:
      pltpu.sync_copy(zb_v.at[pl.ds(0, RPT % CH)],
                      acc_sh.at[pl.ds(r0 + (RPT // CH) * CH, RPT % CH)])

    pltpu.sync_copy(row_hbm.at[pl.ds((c * NS + s) * A_CR, A_CR)], idx_v)
    plsc.subcore_barrier()

    @pl.loop(0, A_CR)
    def _(j):
      pltpu.sync_copy(ones_v, acc_sh.at[idx_v.at[j]], add=True)

    plsc.subcore_barrier()

    @pl.when(c == 0)
    def _():
      pltpu.sync_copy(acc_sh.at[pl.ds(r0, RPT)], deg0_hbm.at[pl.ds(r0, RPT)])

    @pl.when(c == 1)
    def _():
      pltpu.sync_copy(acc_sh.at[pl.ds(r0, RPT)], deg1_hbm.at[pl.ds(r0, RPT)])

  deg0, deg1 = deg_kernel(row2d)

  BLK = 1000

  def proj_body(deg0_ref, deg1_ref, x_ref, w_ref, m0_ref, m1_ref):
    d = lax.rsqrt(deg0_ref[:, :1] + deg1_ref[:, :1])
    h = jnp.dot(x_ref[...], w_ref[...], preferred_element_type=jnp.float32)
    hm = h * d
    m0_ref[...] = hm[:, :H]
    m1_ref[...] = hm[:, H:]

  m0, m1 = pl.pallas_call(
      proj_body,
      grid=(N // BLK,),
      in_specs=[
          pl.BlockSpec((BLK, LANES), lambda i: (i, 0)),
          pl.BlockSpec((BLK, LANES), lambda i: (i, 0)),
          pl.BlockSpec((BLK, D_in), lambda i: (i, 0)),
          pl.BlockSpec((D_in, D_out), lambda i: (0, 0)),
      ],
      out_specs=[
          pl.BlockSpec((BLK, H), lambda i: (i, 0)),
          pl.BlockSpec((BLK, H), lambda i: (i, 0)),
      ],
      out_shape=[jax.ShapeDtypeStruct((N, H), jnp.float32)] * 2,
  )(deg0, deg1, x, W)

  @functools.partial(
      pl.kernel,
      out_type=[jax.ShapeDtypeStruct((NPAD, H), jnp.float32)] * 2,
      mesh=mesh,
      scratch_types=[
          pltpu.VMEM((KB, CH), jnp.int32),
          pltpu.VMEM((KB, CH), jnp.int32),
          pltpu.VMEM((CH, H), jnp.float32),
          pltpu.VMEM((CH, H), jnp.float32),
          pltpu.VMEM_SHARED((NPAD, H), jnp.float32),
          pltpu.SemaphoreType.DMA,
          pltpu.SemaphoreType.DMA,
      ],
  )
  def spmm_kernel(m0_hbm, m1_hbm, row_hbm, col_hbm, out0_hbm, out1_hbm,
                  ridx, cidx, buf_a, buf_b, acc_sh, sem_a, sem_b):
    c = lax.axis_index("c")
    s = lax.axis_index("s")
    r0 = s * RPT

    def pipeline(m_hbm, out_hbm):
      @pl.loop(0, CH)
      def _(i):
        @pl.loop(0, H // LANES)
        def _(k):
          buf_a[i, pl.ds(k * LANES, LANES)] = jnp.zeros((LANES,), jnp.float32)

      @pl.loop(0, RPT // CH)
      def _(k):
        pltpu.sync_copy(buf_a, acc_sh.at[pl.ds(r0 + k * CH, CH)])
      if RPT % CH:
        pltpu.sync_copy(buf_a.at[pl.ds(0, RPT % CH)],
                        acc_sh.at[pl.ds(r0 + (RPT // CH) * CH, RPT % CH)])

      plsc.subcore_barrier()

      @pl.loop(0, C_CR // KB)
      def _(b):
        pltpu.sync_copy(row_hbm.at[pl.ds(s * C_CR + b * KB, KB)], ridx)
        pltpu.sync_copy(col_hbm.at[pl.ds(s * C_CR + b * KB, KB)], cidx)

        pltpu.async_copy(m_hbm.at[cidx.at[0]], buf_a, sem_a)

        @pl.loop(0, KB // 2 - 1)
        def _(i):
          j = 2 * i
          pltpu.make_async_copy(m_hbm.at[cidx.at[j]], buf_a, sem_a).wait()
          pltpu.async_copy(m_hbm.at[cidx.at[j + 1]], buf_b, sem_b)
          pltpu.sync_copy(buf_a, acc_sh.at[ridx.at[j]], add=True)
          pltpu.make_async_copy(m_hbm.at[cidx.at[j + 1]], buf_b, sem_b).wait()
          pltpu.async_copy(m_hbm.at[cidx.at[j + 2]], buf_a, sem_a)
          pltpu.sync_copy(buf_b, acc_sh.at[ridx.at[j + 1]], add=True)

        jl = KB - 2
        pltpu.make_async_copy(m_hbm.at[cidx.at[jl]], buf_a, sem_a).wait()
        pltpu.async_copy(m_hbm.at[cidx.at[jl + 1]], buf_b, sem_b)
        pltpu.sync_copy(buf_a, acc_sh.at[ridx.at[jl]], add=True)
        pltpu.make_async_copy(m_hbm.at[cidx.at[jl + 1]], buf_b, sem_b).wait()
        pltpu.sync_copy(buf_b, acc_sh.at[ridx.at[jl + 1]], add=True)

      plsc.subcore_barrier()
      pltpu.sync_copy(acc_sh.at[pl.ds(r0, RPT)], out_hbm.at[pl.ds(r0, RPT)])

    @pl.when(c == 0)
    def _():
      pipeline(m0_hbm, out0_hbm)

    @pl.when(c == 1)
    def _():
      pipeline(m1_hbm, out1_hbm)

  acc0, acc1 = spmm_kernel(m0, m1, row2d, col2d)

  def final_body(deg0_ref, deg1_ref, a0_ref, a1_ref, o_ref):
    d = lax.rsqrt(deg0_ref[:, :1] + deg1_ref[:, :1])
    o_ref[...] = jnp.concatenate([a0_ref[...] * d, a1_ref[...] * d], axis=1)

  out = pl.pallas_call(
      final_body,
      grid=(N // BLK,),
      in_specs=[
          pl.BlockSpec((BLK, LANES), lambda i: (i, 0)),
          pl.BlockSpec((BLK, LANES), lambda i: (i, 0)),
          pl.BlockSpec((BLK, H), lambda i: (i, 0)),
          pl.BlockSpec((BLK, H), lambda i: (i, 0)),
      ],
      out_specs=pl.BlockSpec((BLK, D_out), lambda i: (i, 0)),
      out_shape=jax.ShapeDtypeStruct((N, D_out), jnp.float32),
  )(deg0, deg1, acc0, acc1)
  return out


def kernel(x, edge_index, W):
  return _gcn(x, edge_index, W)


# speedup vs baseline: 18.3231x; 1.2219x over previous
import functools

import jax
import jax.numpy as jnp
from jax import lax
from jax.experimental import pallas as pl
from jax.experimental.pallas import tpu as pltpu
from jax.experimental.pallas import tpu_sc as plsc

CH = 128
KB = 16
NS = 16
NC = 2
LANES = 16


def _gcn(x, edge_index, W):
  N, D_in = x.shape
  D_out = W.shape[1]
  H = D_out // 2
  E = edge_index.shape[1]

  unit = NC * NS * 8 * CH
  EP = ((E + unit - 1) // unit) * unit
  CROWS = EP // CH
  A_CR = CROWS // (NC * NS)
  C_CR = CROWS // NS
  NPAD = ((N + 64 + 127) // 128) * 128
  RPT = NPAD // NS

  row = edge_index[0].astype(jnp.int32)
  col = edge_index[1].astype(jnp.int32)
  pad_i = jnp.arange(EP - E, dtype=jnp.int32)
  row2d = jnp.concatenate([row, N + (pad_i % 64)]).reshape(CROWS, CH)
  col2d = jnp.concatenate([col, pad_i % N]).reshape(CROWS, CH)

  mesh = plsc.VectorSubcoreMesh(core_axis_name="c", subcore_axis_name="s",
                                num_cores=NC, num_subcores=NS)

  @functools.partial(
      pl.kernel,
      out_type=[jax.ShapeDtypeStruct((NPAD, LANES), jnp.float32)] * 2,
      mesh=mesh,
      scratch_types=[
          pltpu.VMEM((A_CR, CH), jnp.int32),
          pltpu.VMEM((CH, LANES), jnp.float32),
          pltpu.VMEM((CH, LANES), jnp.float32),
          pltpu.VMEM_SHARED((NPAD, LANES), jnp.float32),
      ],
  )
  def deg_kernel(row_hbm, deg0_hbm, deg1_hbm, idx_v, ones_v, zb_v, acc_sh):
    c = lax.axis_index("c")
    s = lax.axis_index("s")
    r0 = s * RPT

    @pl.loop(0, CH)
    def _(i):
      ones_v[i, :] = jnp.ones((LANES,), jnp.float32)
      zb_v[i, :] = jnp.zeros((LANES,), jnp.float32)

    @pl.loop(0, RPT // CH)
    def _(k):
      pltpu.sync_copy(zb_v, acc_sh.at[pl.ds(r0 + k * CH, CH)])
    if RPT % CH:
      pltpu.sync_copy(zb_v.at[pl.ds(0, RPT % CH)],
                      acc_sh.at[pl.ds(r0 + (RPT // CH) * CH, RPT % CH)])

    pltpu.sync_copy(row_hbm.at[pl.ds((c * NS + s) * A_CR, A_CR)], idx_v)
    plsc.subcore_barrier()

    @pl.loop(0, A_CR)
    def _(j):
      pltpu.sync_copy(ones_v, acc_sh.at[idx_v.at[j]], add=True)

    plsc.subcore_barrier()

    @pl.when(c == 0)
    def _():
      pltpu.sync_copy(acc_sh.at[pl.ds(r0, RPT)], deg0_hbm.at[pl.ds(r0, RPT)])

    @pl.when(c == 1)
    def _():
      pltpu.sync_copy(acc_sh.at[pl.ds(r0, RPT)], deg1_hbm.at[pl.ds(r0, RPT)])

  deg0, deg1 = deg_kernel(row2d)

  BLK = 1000

  def proj_body(deg0_ref, deg1_ref, x_ref, w_ref, m0_ref, m1_ref):
    d = lax.rsqrt(deg0_ref[:, :1] + deg1_ref[:, :1])
    h = jnp.dot(x_ref[...], w_ref[...], preferred_element_type=jnp.float32)
    hm = h * d
    m0_ref[...] = hm[:, :H]
    m1_ref[...] = hm[:, H:]

  m0, m1 = pl.pallas_call(
      proj_body,
      grid=(N // BLK,),
      in_specs=[
          pl.BlockSpec((BLK, LANES), lambda i: (i, 0)),
          pl.BlockSpec((BLK, LANES), lambda i: (i, 0)),
          pl.BlockSpec((BLK, D_in), lambda i: (i, 0)),
          pl.BlockSpec((D_in, D_out), lambda i: (0, 0)),
      ],
      out_specs=[
          pl.BlockSpec((BLK, H), lambda i: (i, 0)),
          pl.BlockSpec((BLK, H), lambda i: (i, 0)),
      ],
      out_shape=[jax.ShapeDtypeStruct((N, H), jnp.float32)] * 2,
  )(deg0, deg1, x, W)

  @functools.partial(
      pl.kernel,
      out_type=[jax.ShapeDtypeStruct((NPAD, H), jnp.float32)] * 2,
      mesh=mesh,
      scratch_types=[
          pltpu.VMEM((KB, CH), jnp.int32),
          pltpu.VMEM((KB, CH), jnp.int32),
          pltpu.VMEM((CH, H), jnp.float32),
          pltpu.VMEM((CH, H), jnp.float32),
          pltpu.VMEM_SHARED((NPAD, H), jnp.float32),
          pltpu.SemaphoreType.DMA,
          pltpu.SemaphoreType.DMA,
      ],
  )
  def spmm_kernel(m0_hbm, m1_hbm, row_hbm, col_hbm, out0_hbm, out1_hbm,
                  ridx, cidx, buf_a, buf_b, acc_sh, sem_a, sem_b):
    c = lax.axis_index("c")
    s = lax.axis_index("s")
    r0 = s * RPT

    def pipeline(m_hbm, out_hbm):
      @pl.loop(0, CH)
      def _(i):
        @pl.loop(0, H // LANES)
        def _(k):
          buf_a[i, pl.ds(k * LANES, LANES)] = jnp.zeros((LANES,), jnp.float32)

      @pl.loop(0, RPT // CH)
      def _(k):
        pltpu.sync_copy(buf_a, acc_sh.at[pl.ds(r0 + k * CH, CH)])
      if RPT % CH:
        pltpu.sync_copy(buf_a.at[pl.ds(0, RPT % CH)],
                        acc_sh.at[pl.ds(r0 + (RPT // CH) * CH, RPT % CH)])

      plsc.subcore_barrier()

      @pl.loop(0, C_CR // KB)
      def _(b):
        pltpu.sync_copy(row_hbm.at[pl.ds(s * C_CR + b * KB, KB)], ridx)
        pltpu.sync_copy(col_hbm.at[pl.ds(s * C_CR + b * KB, KB)], cidx)

        pltpu.async_copy(m_hbm.at[cidx.at[0]], buf_a, sem_a)

        @pl.loop(0, KB // 2 - 1)
        def _(i):
          j = 2 * i
          pltpu.make_async_copy(m_hbm.at[cidx.at[j]], buf_a, sem_a).wait()
          pltpu.async_copy(m_hbm.at[cidx.at[j + 1]], buf_b, sem_b)
          pltpu.sync_copy(buf_a, acc_sh.at[ridx.at[j]], add=True)
          pltpu.make_async_copy(m_hbm.at[cidx.at[j + 1]], buf_b, sem_b).wait()
          pltpu.async_copy(m_hbm.at[cidx.at[j + 2]], buf_a, sem_a)
          pltpu.sync_copy(buf_b, acc_sh.at[ridx.at[j + 1]], add=True)

        jl = KB - 2
        pltpu.make_async_copy(m_hbm.at[cidx.at[jl]], buf_a, sem_a).wait()
        pltpu.async_copy(m_hbm.at[cidx.at[jl + 1]], buf_b, sem_b)
        pltpu.sync_copy(buf_a, acc_sh.at[ridx.at[jl]], add=True)
        pltpu.make_async_copy(m_hbm.at[cidx.at[jl + 1]], buf_b, sem_b).wait()
        pltpu.sync_copy(buf_b, acc_sh.at[ridx.at[jl + 1]], add=True)

      plsc.subcore_barrier()
      pltpu.sync_copy(acc_sh.at[pl.ds(r0, RPT)], out_hbm.at[pl.ds(r0, RPT)])

    @pl.when(c == 0)
    def _():
      pipeline(m0_hbm, out0_hbm)

    @pl.when(c == 1)
    def _():
      pipeline(m1_hbm, out1_hbm)

  acc0, acc1 = spmm_kernel(m0, m1, row2d, col2d)

  def final_body(deg0_ref, deg1_ref, a0_ref, a1_ref, o_ref):
    d = lax.rsqrt(deg0_ref[:, :1] + deg1_ref[:, :1])
    o_ref[...] = jnp.concatenate([a0_ref[...] * d, a1_ref[...] * d], axis=1)

  out = pl.pallas_call(
      final_body,
      grid=(N // BLK,),
      in_specs=[
          pl.BlockSpec((BLK, LANES), lambda i: (i, 0)),
          pl.BlockSpec((BLK, LANES), lambda i: (i, 0)),
          pl.BlockSpec((BLK, H), lambda i: (i, 0)),
          pl.BlockSpec((BLK, H), lambda i: (i, 0)),
      ],
      out_specs=pl.BlockSpec((BLK, D_out), lambda i: (i, 0)),
      out_shape=jax.ShapeDtypeStruct((N, D_out), jnp.float32),
  )(deg0, deg1, acc0, acc1)
  return out


def kernel(x, edge_index, W):
  return _gcn(x, edge_index, W)


# speedup vs baseline: 18.8846x; 1.0306x over previous
import functools

import jax
import jax.numpy as jnp
from jax import lax
from jax.experimental import pallas as pl
from jax.experimental.pallas import tpu as pltpu
from jax.experimental.pallas import tpu_sc as plsc

CH = 128
KB = 16
NS = 16
NC = 2
LANES = 16


def _gcn(x, edge_index, W):
  N, D_in = x.shape
  D_out = W.shape[1]
  H = D_out // 2
  E = edge_index.shape[1]

  unit = NC * NS * 8 * CH
  EP = ((E + unit - 1) // unit) * unit
  CROWS = EP // CH
  A_CR = CROWS // (NC * NS)
  C_CR = CROWS // NS
  NPAD = ((N + 64 + 127) // 128) * 128
  RPT = NPAD // NS

  row = edge_index[0].astype(jnp.int32)
  col = edge_index[1].astype(jnp.int32)
  pad_i = jnp.arange(EP - E, dtype=jnp.int32)
  row2d = jnp.concatenate([row, N + (pad_i % 64)]).reshape(CROWS, CH)
  col2d = jnp.concatenate([col, pad_i % N]).reshape(CROWS, CH)

  mesh = plsc.VectorSubcoreMesh(core_axis_name="c", subcore_axis_name="s",
                                num_cores=NC, num_subcores=NS)

  @functools.partial(
      pl.kernel,
      out_type=[jax.ShapeDtypeStruct((NPAD, LANES), jnp.float32)] * 2,
      mesh=mesh,
      scratch_types=[
          pltpu.VMEM((A_CR, CH), jnp.int32),
          pltpu.VMEM((CH, LANES), jnp.float32),
          pltpu.VMEM((CH, LANES), jnp.float32),
          pltpu.VMEM_SHARED((NPAD, LANES), jnp.float32),
      ],
  )
  def deg_kernel(row_hbm, deg0_hbm, deg1_hbm, idx_v, ones_v, zb_v, acc_sh):
    c = lax.axis_index("c")
    s = lax.axis_index("s")
    r0 = s * RPT

    @pl.loop(0, CH)
    def _(i):
      ones_v[i, :] = jnp.ones((LANES,), jnp.float32)
      zb_v[i, :] = jnp.zeros((LANES,), jnp.float32)

    @pl.loop(0, RPT // CH)
    def _(k):
      pltpu.sync_copy(zb_v, acc_sh.at[pl.ds(r0 + k * CH, CH)])
    if RPT % CH:
      pltpu.sync_copy(zb_v.at[pl.ds(0, RPT % CH)],
                      acc_sh.at[pl.ds(r0 + (RPT // CH) * CH, RPT % CH)])

    pltpu.sync_copy(row_hbm.at[pl.ds((c * NS + s) * A_CR, A_CR)], idx_v)
    plsc.subcore_barrier()

    @pl.loop(0, A_CR)
    def _(j):
      pltpu.sync_copy(ones_v, acc_sh.at[idx_v.at[j]], add=True)

    plsc.subcore_barrier()

    @pl.when(c == 0)
    def _():
      pltpu.sync_copy(acc_sh.at[pl.ds(r0, RPT)], deg0_hbm.at[pl.ds(r0, RPT)])

    @pl.when(c == 1)
    def _():
      pltpu.sync_copy(acc_sh.at[pl.ds(r0, RPT)], deg1_hbm.at[pl.ds(r0, RPT)])

  deg0, deg1 = deg_kernel(row2d)

  BLK = 1000

  def proj_body(deg0_ref, deg1_ref, x_ref, w_ref, m0_ref, m1_ref):
    d = lax.rsqrt(deg0_ref[:, :1] + deg1_ref[:, :1])
    h = jnp.dot(x_ref[...], w_ref[...], preferred_element_type=jnp.float32)
    hm = h * d
    m0_ref[...] = hm[:, :H]
    m1_ref[...] = hm[:, H:]

  m0, m1 = pl.pallas_call(
      proj_body,
      grid=(N // BLK,),
      in_specs=[
          pl.BlockSpec((BLK, LANES), lambda i: (i, 0)),
          pl.BlockSpec((BLK, LANES), lambda i: (i, 0)),
          pl.BlockSpec((BLK, D_in), lambda i: (i, 0)),
          pl.BlockSpec((D_in, D_out), lambda i: (0, 0)),
      ],
      out_specs=[
          pl.BlockSpec((BLK, H), lambda i: (i, 0)),
          pl.BlockSpec((BLK, H), lambda i: (i, 0)),
      ],
      out_shape=[jax.ShapeDtypeStruct((N, H), jnp.float32)] * 2,
  )(deg0, deg1, x, W)

  @functools.partial(
      pl.kernel,
      out_type=[jax.ShapeDtypeStruct((NPAD, H), jnp.float32)] * 2,
      mesh=mesh,
      scratch_types=[
          pltpu.VMEM((KB, CH), jnp.int32),
          pltpu.VMEM((KB, CH), jnp.int32),
          pltpu.VMEM((KB, CH), jnp.int32),
          pltpu.VMEM((KB, CH), jnp.int32),
          pltpu.VMEM((CH, H), jnp.float32),
          pltpu.VMEM((CH, H), jnp.float32),
          pltpu.VMEM_SHARED((NPAD, H), jnp.float32),
          pltpu.SemaphoreType.DMA,
          pltpu.SemaphoreType.DMA,
          pltpu.SemaphoreType.DMA,
          pltpu.SemaphoreType.DMA,
      ],
  )
  def spmm_kernel(m0_hbm, m1_hbm, row_hbm, col_hbm, out0_hbm, out1_hbm,
                  ri0, ci0, ri1, ci1, buf_a, buf_b, acc_sh,
                  sem_a, sem_b, sem_i0, sem_i1):
    c = lax.axis_index("c")
    s = lax.axis_index("s")
    r0 = s * RPT
    NB = C_CR // KB
    parity = ((ri0, ci0, sem_i0), (ri1, ci1, sem_i1))

    def idx_fetch(b, ri, ci, sem_i):
      pltpu.async_copy(row_hbm.at[pl.ds(s * C_CR + b * KB, KB)], ri, sem_i)
      pltpu.async_copy(col_hbm.at[pl.ds(s * C_CR + b * KB, KB)], ci, sem_i)

    def pipeline(m_hbm, out_hbm):
      for p in range(2):
        idx_fetch(p, *parity[p])

      @pl.loop(0, CH)
      def _(i):
        @pl.loop(0, H // LANES)
        def _(k):
          buf_a[i, pl.ds(k * LANES, LANES)] = jnp.zeros((LANES,), jnp.float32)

      @pl.loop(0, RPT // CH)
      def _(k):
        pltpu.sync_copy(buf_a, acc_sh.at[pl.ds(r0 + k * CH, CH)])
      if RPT % CH:
        pltpu.sync_copy(buf_a.at[pl.ds(0, RPT % CH)],
                        acc_sh.at[pl.ds(r0 + (RPT // CH) * CH, RPT % CH)])

      plsc.subcore_barrier()

      @pl.loop(0, NB // 2)
      def _(sb):
        for p in range(2):
          ri, ci, sem_i = parity[p]
          b = 2 * sb + p
          pltpu.make_async_copy(
              row_hbm.at[pl.ds(s * C_CR + b * KB, KB)], ri, sem_i).wait()
          pltpu.make_async_copy(
              col_hbm.at[pl.ds(s * C_CR + b * KB, KB)], ci, sem_i).wait()

          pltpu.async_copy(m_hbm.at[ci.at[0]], buf_a, sem_a)

          @pl.loop(0, KB // 2 - 1)
          def _(i):
            j = 2 * i
            pltpu.make_async_copy(m_hbm.at[ci.at[j]], buf_a, sem_a).wait()
            pltpu.async_copy(m_hbm.at[ci.at[j + 1]], buf_b, sem_b)
            pltpu.sync_copy(buf_a, acc_sh.at[ri.at[j]], add=True)
            pltpu.make_async_copy(m_hbm.at[ci.at[j + 1]], buf_b, sem_b).wait()
            pltpu.async_copy(m_hbm.at[ci.at[j + 2]], buf_a, sem_a)
            pltpu.sync_copy(buf_b, acc_sh.at[ri.at[j + 1]], add=True)

          jl = KB - 2
          pltpu.make_async_copy(m_hbm.at[ci.at[jl]], buf_a, sem_a).wait()
          pltpu.async_copy(m_hbm.at[ci.at[jl + 1]], buf_b, sem_b)
          pltpu.sync_copy(buf_a, acc_sh.at[ri.at[jl]], add=True)
          pltpu.make_async_copy(m_hbm.at[ci.at[jl + 1]], buf_b, sem_b).wait()
          pltpu.sync_copy(buf_b, acc_sh.at[ri.at[jl + 1]], add=True)

          @pl.when(b + 2 < NB)
          def _():
            idx_fetch(b + 2, ri, ci, sem_i)

      plsc.subcore_barrier()
      pltpu.sync_copy(acc_sh.at[pl.ds(r0, RPT)], out_hbm.at[pl.ds(r0, RPT)])

    @pl.when(c == 0)
    def _():
      pipeline(m0_hbm, out0_hbm)

    @pl.when(c == 1)
    def _():
      pipeline(m1_hbm, out1_hbm)

  acc0, acc1 = spmm_kernel(m0, m1, row2d, col2d)

  def final_body(deg0_ref, deg1_ref, a0_ref, a1_ref, o_ref):
    d = lax.rsqrt(deg0_ref[:, :1] + deg1_ref[:, :1])
    o_ref[...] = jnp.concatenate([a0_ref[...] * d, a1_ref[...] * d], axis=1)

  out = pl.pallas_call(
      final_body,
      grid=(N // BLK,),
      in_specs=[
          pl.BlockSpec((BLK, LANES), lambda i: (i, 0)),
          pl.BlockSpec((BLK, LANES), lambda i: (i, 0)),
          pl.BlockSpec((BLK, H), lambda i: (i, 0)),
          pl.BlockSpec((BLK, H), lambda i: (i, 0)),
      ],
      out_specs=pl.BlockSpec((BLK, D_out), lambda i: (i, 0)),
      out_shape=jax.ShapeDtypeStruct((N, D_out), jnp.float32),
  )(deg0, deg1, acc0, acc1)
  return out


def kernel(x, edge_index, W):
  return _gcn(x, edge_index, W)


# speedup vs baseline: 20.6053x; 1.0911x over previous
import functools

import jax
import jax.numpy as jnp
from jax import lax
from jax.experimental import pallas as pl
from jax.experimental.pallas import tpu as pltpu
from jax.experimental.pallas import tpu_sc as plsc

CH = 128
CHA = 64
KB = 8
NS = 16
NC = 2
LANES = 16


def _gcn(x, edge_index, W):
  N, D_in = x.shape
  D_out = W.shape[1]
  H = D_out // 2
  E = edge_index.shape[1]

  unit = NS * 8 * CH
  EP = ((E + unit - 1) // unit) * unit
  ACROWS = EP // CHA
  A_CR = ACROWS // (NC * NS)
  CCROWS = EP // CH
  C_CR = CCROWS // NS
  NB = C_CR // KB
  NPAD = ((N + 64 + 127) // 128) * 128
  RPT = NPAD // NS

  row = edge_index[0].astype(jnp.int32)
  col = edge_index[1].astype(jnp.int32)
  pad_i = jnp.arange(EP - E, dtype=jnp.int32)
  row_p = jnp.concatenate([row, N + (pad_i % 64)])
  col_p = jnp.concatenate([col, pad_i % N])
  rowA2d = row_p.reshape(ACROWS, CHA)
  rowC2d = row_p.reshape(CCROWS, CH)
  colC2d = col_p.reshape(CCROWS, CH)

  mesh = plsc.VectorSubcoreMesh(core_axis_name="c", subcore_axis_name="s",
                                num_cores=NC, num_subcores=NS)

  @functools.partial(
      pl.kernel,
      out_type=[jax.ShapeDtypeStruct((NPAD, LANES), jnp.float32)] * 2,
      mesh=mesh,
      scratch_types=[
          pltpu.VMEM((A_CR, CHA), jnp.int32),
          pltpu.VMEM((CHA, LANES), jnp.float32),
          pltpu.VMEM((CHA, LANES), jnp.float32),
          pltpu.VMEM_SHARED((NPAD, LANES), jnp.float32),
      ],
  )
  def deg_kernel(row_hbm, deg0_hbm, deg1_hbm, idx_v, ones_v, zb_v, acc_sh):
    c = lax.axis_index("c")
    s = lax.axis_index("s")
    r0 = s * RPT

    @pl.loop(0, CHA)
    def _(i):
      ones_v[i, :] = jnp.ones((LANES,), jnp.float32)
      zb_v[i, :] = jnp.zeros((LANES,), jnp.float32)

    @pl.loop(0, RPT // CHA)
    def _(k):
      pltpu.sync_copy(zb_v, acc_sh.at[pl.ds(r0 + k * CHA, CHA)])
    if RPT % CHA:
      pltpu.sync_copy(zb_v.at[pl.ds(0, RPT % CHA)],
                      acc_sh.at[pl.ds(r0 + (RPT // CHA) * CHA, RPT % CHA)])

    pltpu.sync_copy(row_hbm.at[pl.ds((c * NS + s) * A_CR, A_CR)], idx_v)
    plsc.subcore_barrier()

    @pl.loop(0, A_CR)
    def _(j):
      pltpu.sync_copy(ones_v, acc_sh.at[idx_v.at[j]], add=True)

    plsc.subcore_barrier()

    @pl.when(c == 0)
    def _():
      pltpu.sync_copy(acc_sh.at[pl.ds(r0, RPT)], deg0_hbm.at[pl.ds(r0, RPT)])

    @pl.when(c == 1)
    def _():
      pltpu.sync_copy(acc_sh.at[pl.ds(r0, RPT)], deg1_hbm.at[pl.ds(r0, RPT)])

  deg0, deg1 = deg_kernel(rowA2d)

  BLK = 1000

  def proj_body(deg0_ref, deg1_ref, x_ref, w_ref, m0_ref, m1_ref):
    d = lax.rsqrt(deg0_ref[:, :1] + deg1_ref[:, :1])
    h = jnp.dot(x_ref[...], w_ref[...], preferred_element_type=jnp.float32)
    hm = h * d
    m0_ref[...] = hm[:, :H]
    m1_ref[...] = hm[:, H:]

  m0, m1 = pl.pallas_call(
      proj_body,
      grid=(N // BLK,),
      in_specs=[
          pl.BlockSpec((BLK, LANES), lambda i: (i, 0)),
          pl.BlockSpec((BLK, LANES), lambda i: (i, 0)),
          pl.BlockSpec((BLK, D_in), lambda i: (i, 0)),
          pl.BlockSpec((D_in, D_out), lambda i: (0, 0)),
      ],
      out_specs=[
          pl.BlockSpec((BLK, H), lambda i: (i, 0)),
          pl.BlockSpec((BLK, H), lambda i: (i, 0)),
      ],
      out_shape=[jax.ShapeDtypeStruct((N, H), jnp.float32)] * 2,
  )(deg0, deg1, x, W)

  @functools.partial(
      pl.kernel,
      out_type=[jax.ShapeDtypeStruct((NPAD, H), jnp.float32)] * 2,
      mesh=mesh,
      scratch_types=[
          pltpu.VMEM((KB, CH), jnp.int32),
          pltpu.VMEM((KB, CH), jnp.int32),
          pltpu.VMEM((KB, CH), jnp.int32),
          pltpu.VMEM((KB, CH), jnp.int32),
          pltpu.VMEM((CH, H), jnp.float32),
          pltpu.VMEM((CH, H), jnp.float32),
          pltpu.VMEM_SHARED((NPAD, H), jnp.float32),
          pltpu.SemaphoreType.DMA,
          pltpu.SemaphoreType.DMA,
          pltpu.SemaphoreType.DMA,
          pltpu.SemaphoreType.DMA,
      ],
  )
  def spmm_kernel(m0_hbm, m1_hbm, row_hbm, col_hbm, out0_hbm, out1_hbm,
                  ri0, ci0, ri1, ci1, buf_a, buf_b, acc_sh,
                  sem_a, sem_b, sem_i0, sem_i1):
    c = lax.axis_index("c")
    s = lax.axis_index("s")
    r0 = s * RPT
    NB = C_CR // KB
    parity = ((ri0, ci0, sem_i0), (ri1, ci1, sem_i1))

    def idx_fetch(b, ri, ci, sem_i):
      pltpu.async_copy(row_hbm.at[pl.ds(s * C_CR + b * KB, KB)], ri, sem_i)
      pltpu.async_copy(col_hbm.at[pl.ds(s * C_CR + b * KB, KB)], ci, sem_i)

    def pipeline(m_hbm, out_hbm):
      for p in range(2):
        idx_fetch(p, *parity[p])

      @pl.loop(0, CH)
      def _(i):
        @pl.loop(0, H // LANES)
        def _(k):
          buf_a[i, pl.ds(k * LANES, LANES)] = jnp.zeros((LANES,), jnp.float32)

      @pl.loop(0, RPT // CH)
      def _(k):
        pltpu.sync_copy(buf_a, acc_sh.at[pl.ds(r0 + k * CH, CH)])
      if RPT % CH:
        pltpu.sync_copy(buf_a.at[pl.ds(0, RPT % CH)],
                        acc_sh.at[pl.ds(r0 + (RPT // CH) * CH, RPT % CH)])

      plsc.subcore_barrier()

      @pl.loop(0, NB // 2)
      def _(sb):
        for p in range(2):
          ri, ci, sem_i = parity[p]
          b = 2 * sb + p
          pltpu.make_async_copy(
              row_hbm.at[pl.ds(s * C_CR + b * KB, KB)], ri, sem_i).wait()
          pltpu.make_async_copy(
              col_hbm.at[pl.ds(s * C_CR + b * KB, KB)], ci, sem_i).wait()

          pltpu.async_copy(m_hbm.at[ci.at[0]], buf_a, sem_a)

          @pl.loop(0, KB // 2 - 1)
          def _(i):
            j = 2 * i
            pltpu.make_async_copy(m_hbm.at[ci.at[j]], buf_a, sem_a).wait()
            pltpu.async_copy(m_hbm.at[ci.at[j + 1]], buf_b, sem_b)
            pltpu.sync_copy(buf_a, acc_sh.at[ri.at[j]], add=True)
            pltpu.make_async_copy(m_hbm.at[ci.at[j + 1]], buf_b, sem_b).wait()
            pltpu.async_copy(m_hbm.at[ci.at[j + 2]], buf_a, sem_a)
            pltpu.sync_copy(buf_b, acc_sh.at[ri.at[j + 1]], add=True)

          jl = KB - 2
          pltpu.make_async_copy(m_hbm.at[ci.at[jl]], buf_a, sem_a).wait()
          pltpu.async_copy(m_hbm.at[ci.at[jl + 1]], buf_b, sem_b)
          pltpu.sync_copy(buf_a, acc_sh.at[ri.at[jl]], add=True)
          pltpu.make_async_copy(m_hbm.at[ci.at[jl + 1]], buf_b, sem_b).wait()
          pltpu.sync_copy(buf_b, acc_sh.at[ri.at[jl + 1]], add=True)

          @pl.when(b + 2 < NB)
          def _():
            idx_fetch(b + 2, ri, ci, sem_i)

      plsc.subcore_barrier()
      pltpu.sync_copy(acc_sh.at[pl.ds(r0, RPT)], out_hbm.at[pl.ds(r0, RPT)])

    @pl.when(c == 0)
    def _():
      pipeline(m0_hbm, out0_hbm)

    @pl.when(c == 1)
    def _():
      pipeline(m1_hbm, out1_hbm)

  acc0, acc1 = spmm_kernel(m0, m1, rowC2d, colC2d)

  def final_body(deg0_ref, deg1_ref, a0_ref, a1_ref, o_ref):
    d = lax.rsqrt(deg0_ref[:, :1] + deg1_ref[:, :1])
    o_ref[...] = jnp.concatenate([a0_ref[...] * d, a1_ref[...] * d], axis=1)

  out = pl.pallas_call(
      final_body,
      grid=(N // BLK,),
      in_specs=[
          pl.BlockSpec((BLK, LANES), lambda i: (i, 0)),
          pl.BlockSpec((BLK, LANES), lambda i: (i, 0)),
          pl.BlockSpec((BLK, H), lambda i: (i, 0)),
          pl.BlockSpec((BLK, H), lambda i: (i, 0)),
      ],
      out_specs=pl.BlockSpec((BLK, D_out), lambda i: (i, 0)),
      out_shape=jax.ShapeDtypeStruct((N, D_out), jnp.float32),
  )(deg0, deg1, acc0, acc1)
  return out


def kernel(x, edge_index, W):
  return _gcn(x, edge_index, W)
